# Initial kernel scaffold; baseline (speedup 1.0000x reference)
import jax
import jax.numpy as jnp
from jax.experimental import pallas as pl


def kernel(pos, PE_weight):
    raise NotImplementedError("write your pallas kernel here")


# speedup vs baseline: 1.3228x; 1.3228x over previous
import functools

import jax
import jax.numpy as jnp
from jax import lax
from jax.experimental import pallas as pl
from jax.experimental.pallas import tpu as pltpu
from jax.experimental.pallas import tpu_sc as plsc

_MODEL_DIM = 2048
_NUM_CORES = 2
_NUM_SUBCORES = 16
_NUM_WORKERS = _NUM_CORES * _NUM_SUBCORES
_CHUNK = 16


def _gather_body(table_hbm, idx_hbm, out_hbm, idx_v, rows_v, sem):
    b_per_w = idx_v.shape[0]
    wid = lax.axis_index("s") * _NUM_CORES + lax.axis_index("c")
    base = wid * b_per_w
    pltpu.sync_copy(idx_hbm.at[pl.ds(base, b_per_w)], idx_v)

    def chunk_step(i, _):
        off = i * _CHUNK
        pltpu.async_copy(
            table_hbm.at[idx_v.at[pl.ds(off, _CHUNK)]], rows_v, sem
        ).wait()
        pltpu.sync_copy(rows_v, out_hbm.at[pl.ds(base + off, _CHUNK)])
        return 0

    lax.fori_loop(0, b_per_w // _CHUNK, chunk_step, 0)


@functools.partial(jax.jit, static_argnames=("total",))
def _sc_gather(table, idx_flat, total):
    b_per_w = total // _NUM_WORKERS
    mesh = plsc.VectorSubcoreMesh(core_axis_name="c", subcore_axis_name="s")
    k = functools.partial(
        pl.kernel,
        mesh=mesh,
        out_type=jax.ShapeDtypeStruct((total, _MODEL_DIM), jnp.float32),
        scratch_types=[
            pltpu.VMEM((b_per_w,), jnp.int32),
            pltpu.VMEM((_CHUNK, _MODEL_DIM), jnp.float32),
            pltpu.SemaphoreType.DMA,
        ],
    )(_gather_body)
    return k(table, idx_flat)


def kernel(pos, PE_weight):
    batch, seq_len = pos.shape
    total = batch * seq_len
    idx_flat = pos.reshape((total,)).astype(jnp.int32)
    out = _sc_gather(PE_weight, idx_flat, total)
    return out.reshape((batch, seq_len, _MODEL_DIM))


# speedup vs baseline: 1.6129x; 1.2193x over previous
import functools

import jax
import jax.numpy as jnp
from jax import lax
from jax.experimental import pallas as pl
from jax.experimental.pallas import tpu as pltpu
from jax.experimental.pallas import tpu_sc as plsc

_MODEL_DIM = 2048
_NUM_CORES = 2
_NUM_SUBCORES = 16
_NUM_WORKERS = _NUM_CORES * _NUM_SUBCORES
_CHUNK = 16


_NBUF = 2


def _gather_body(table_hbm, idx_hbm, out_hbm, idx_v, rows_v, sem0, sem1):
    b_per_w = idx_v.shape[0]
    nchunks = b_per_w // _CHUNK
    sems = (sem0, sem1)
    wid = lax.axis_index("s") * _NUM_CORES + lax.axis_index("c")
    base = wid * b_per_w
    pltpu.sync_copy(idx_hbm.at[pl.ds(base, b_per_w)], idx_v)

    def fire(chunk, buf):
        return pltpu.async_copy(
            table_hbm.at[idx_v.at[pl.ds(chunk * _CHUNK, _CHUNK)]],
            rows_v.at[buf],
            sems[buf],
        )

    for b in range(_NBUF):
        fire(b, b)

    def step(i, _):
        for b in range(_NBUF):
            g = i * _NBUF + b
            pltpu.make_async_copy(
                table_hbm.at[idx_v.at[pl.ds(0, _CHUNK)]], rows_v.at[b], sems[b]
            ).wait()
            pltpu.sync_copy(
                rows_v.at[b], out_hbm.at[pl.ds(base + g * _CHUNK, _CHUNK)]
            )

            @pl.when(g + _NBUF < nchunks)
            def _():
                fire(g + _NBUF, b)

        return 0

    lax.fori_loop(0, nchunks // _NBUF, step, 0)


@functools.partial(jax.jit, static_argnames=("total",))
def _sc_gather(table, idx_flat, total):
    b_per_w = total // _NUM_WORKERS
    mesh = plsc.VectorSubcoreMesh(core_axis_name="c", subcore_axis_name="s")
    k = functools.partial(
        pl.kernel,
        mesh=mesh,
        out_type=jax.ShapeDtypeStruct((total, _MODEL_DIM), jnp.float32),
        scratch_types=[
            pltpu.VMEM((b_per_w,), jnp.int32),
            pltpu.VMEM((_NBUF, _CHUNK, _MODEL_DIM), jnp.float32),
            pltpu.SemaphoreType.DMA,
            pltpu.SemaphoreType.DMA,
        ],
    )(_gather_body)
    return k(table, idx_flat)


def kernel(pos, PE_weight):
    batch, seq_len = pos.shape
    total = batch * seq_len
    idx_flat = pos.reshape((total,)).astype(jnp.int32)
    out = _sc_gather(PE_weight, idx_flat, total)
    return out.reshape((batch, seq_len, _MODEL_DIM))
